# baseline (speedup 1.0000x reference)
import functools

import jax
import jax.numpy as jnp
from jax import lax
from jax.experimental import pallas as pl
from jax.experimental.pallas import tpu as pltpu
from jax.experimental.pallas import tpu_sc as plsc

KEY_DIM = 256
N_E = 1024
E_DIM = 256
KT = 1.0
EPS = 1e-12

BLK = 2304


def _row_normalize(x):
    n = jnp.sqrt(jnp.sum(x * x, axis=-1, keepdims=True))
    return x / jnp.maximum(n, EPS)


def _main_body(x_ref, keys_ref, vparams_ref, idx_ref, vw_ref, kn_ref, vn_ref):
    @pl.when(pl.program_id(0) == 0)
    def _():
        kn_ref[...] = _row_normalize(keys_ref[...])
        vn_ref[...] = _row_normalize(vparams_ref[...])

    xn = _row_normalize(x_ref[...])
    scores = lax.dot_general(
        xn, kn_ref[...],
        (((1,), (1,)), ((), ())),
        preferred_element_type=jnp.float32,
    )
    idx_ref[...] = jnp.argmax(scores, axis=-1).astype(jnp.int32)[:, None]
    e = jnp.exp(scores)
    vw = jnp.dot(e, vn_ref[...], preferred_element_type=jnp.float32)
    vw_ref[...] = _row_normalize(vw)


def _sc_gather(table, idx):
    bt = idx.shape[0]
    info = plsc.get_sparse_core_info()
    nw = info.num_cores * info.num_subcores
    b_per_w = bt // nw
    mesh = plsc.VectorSubcoreMesh(core_axis_name="c", subcore_axis_name="s")

    @functools.partial(
        pl.kernel,
        mesh=mesh,
        out_type=jax.ShapeDtypeStruct((bt, E_DIM), jnp.float32),
        scratch_types=[
            pltpu.VMEM((b_per_w,), jnp.int32),
            pltpu.VMEM((b_per_w, E_DIM), jnp.float32),
            pltpu.SemaphoreType.DMA,
        ],
    )
    def gather_kernel(table_hbm, idx_hbm, out_hbm, idx_v, rows_v, sem):
        wid = lax.axis_index("s") * info.num_cores + lax.axis_index("c")
        base = wid * b_per_w
        pltpu.sync_copy(idx_hbm.at[pl.ds(base, b_per_w)], idx_v)
        pltpu.async_copy(table_hbm.at[idx_v], rows_v, sem).wait()
        pltpu.sync_copy(rows_v, out_hbm.at[pl.ds(base, b_per_w)])

    return gather_kernel(table, idx)


def kernel(key_soft, keys, vparams):
    b, t, _ = key_soft.shape
    bt = b * t
    x = key_soft.reshape(bt, KEY_DIM)

    grid = bt // BLK
    idx3, vw, _, vn = pl.pallas_call(
        _main_body,
        grid=(grid,),
        in_specs=[
            pl.BlockSpec((BLK, KEY_DIM), lambda i: (i, 0)),
            pl.BlockSpec((N_E, KEY_DIM), lambda i: (0, 0)),
            pl.BlockSpec((N_E, E_DIM), lambda i: (0, 0)),
        ],
        out_specs=(
            pl.BlockSpec((BLK, 1), lambda i: (i, 0)),
            pl.BlockSpec((BLK, E_DIM), lambda i: (i, 0)),
            pl.BlockSpec((N_E, KEY_DIM), lambda i: (0, 0)),
            pl.BlockSpec((N_E, E_DIM), lambda i: (0, 0)),
        ),
        out_shape=(
            jax.ShapeDtypeStruct((bt, 1), jnp.int32),
            jax.ShapeDtypeStruct((bt, E_DIM), jnp.float32),
            jax.ShapeDtypeStruct((N_E, KEY_DIM), jnp.float32),
            jax.ShapeDtypeStruct((N_E, E_DIM), jnp.float32),
        ),
    )(x, keys, vparams)
    idx = idx3.reshape(bt)

    vh = _sc_gather(vn, idx)

    return (
        idx.reshape(b, t),
        vw.reshape(b, t, E_DIM),
        vh.reshape(b, t, E_DIM),
    )


# speedup vs baseline: 1.1827x; 1.1827x over previous
import functools

import jax
import jax.numpy as jnp
from jax import lax
from jax.experimental import pallas as pl
from jax.experimental.pallas import tpu as pltpu
from jax.experimental.pallas import tpu_sc as plsc

KEY_DIM = 256
N_E = 1024
E_DIM = 256
KT = 1.0
EPS = 1e-12

BLK = 2304


def _row_normalize(x):
    n = jnp.sqrt(jnp.sum(x * x, axis=-1, keepdims=True))
    return x / jnp.maximum(n, EPS)


def _main_body(x_ref, keys_ref, vparams_ref, idx_ref, vw_ref, kn_ref, vn_ref):
    @pl.when(pl.program_id(0) == 0)
    def _():
        kn_ref[...] = _row_normalize(keys_ref[...])
        vn_ref[...] = _row_normalize(vparams_ref[...])

    xn = _row_normalize(x_ref[...])
    scores = lax.dot_general(
        xn, kn_ref[...],
        (((1,), (1,)), ((), ())),
        preferred_element_type=jnp.float32,
    )
    m = jnp.max(scores, axis=-1, keepdims=True)
    col = lax.broadcasted_iota(jnp.int32, scores.shape, 1)
    idx_ref[...] = jnp.min(
        jnp.where(scores == m, col, jnp.int32(2**30)), axis=-1, keepdims=True
    )
    e = jnp.exp(scores)
    vw = jnp.dot(e, vn_ref[...], preferred_element_type=jnp.float32)
    vw_ref[...] = _row_normalize(vw)


def _sc_gather(table, idx):
    bt = idx.shape[0]
    info = plsc.get_sparse_core_info()
    nw = info.num_cores * info.num_subcores
    b_per_w = bt // nw
    mesh = plsc.VectorSubcoreMesh(core_axis_name="c", subcore_axis_name="s")

    @functools.partial(
        pl.kernel,
        mesh=mesh,
        out_type=jax.ShapeDtypeStruct((bt, E_DIM), jnp.float32),
        scratch_types=[
            pltpu.VMEM((b_per_w,), jnp.int32),
            pltpu.VMEM((b_per_w, E_DIM), jnp.float32),
            pltpu.SemaphoreType.DMA,
        ],
    )
    def gather_kernel(table_hbm, idx_hbm, out_hbm, idx_v, rows_v, sem):
        wid = lax.axis_index("s") * info.num_cores + lax.axis_index("c")
        base = wid * b_per_w
        pltpu.sync_copy(idx_hbm.at[pl.ds(base, b_per_w)], idx_v)
        pltpu.async_copy(table_hbm.at[idx_v], rows_v, sem).wait()
        pltpu.sync_copy(rows_v, out_hbm.at[pl.ds(base, b_per_w)])

    return gather_kernel(table, idx)


def kernel(key_soft, keys, vparams):
    b, t, _ = key_soft.shape
    bt = b * t
    x = key_soft.reshape(bt, KEY_DIM)

    grid = bt // BLK
    idx3, vw, _, vn = pl.pallas_call(
        _main_body,
        grid=(grid,),
        in_specs=[
            pl.BlockSpec((BLK, KEY_DIM), lambda i: (i, 0)),
            pl.BlockSpec((N_E, KEY_DIM), lambda i: (0, 0)),
            pl.BlockSpec((N_E, E_DIM), lambda i: (0, 0)),
        ],
        out_specs=(
            pl.BlockSpec((BLK, 1), lambda i: (i, 0)),
            pl.BlockSpec((BLK, E_DIM), lambda i: (i, 0)),
            pl.BlockSpec((N_E, KEY_DIM), lambda i: (0, 0)),
            pl.BlockSpec((N_E, E_DIM), lambda i: (0, 0)),
        ),
        out_shape=(
            jax.ShapeDtypeStruct((bt, 1), jnp.int32),
            jax.ShapeDtypeStruct((bt, E_DIM), jnp.float32),
            jax.ShapeDtypeStruct((N_E, KEY_DIM), jnp.float32),
            jax.ShapeDtypeStruct((N_E, E_DIM), jnp.float32),
        ),
    )(x, keys, vparams)
    idx = idx3.reshape(bt)

    vh = _sc_gather(vn, idx)

    return (
        idx.reshape(b, t),
        vw.reshape(b, t, E_DIM),
        vh.reshape(b, t, E_DIM),
    )
